# Initial kernel scaffold; baseline (speedup 1.0000x reference)
import jax
import jax.numpy as jnp
from jax.experimental import pallas as pl


def kernel(x, knn_edge_index, ppi_edge_index, cols_Wl, cols_bl, cols_Wr, rows_Wl, rows_bl, rows_Wr):
    raise NotImplementedError("write your pallas kernel here")


# speedup vs baseline: 14.8570x; 14.8570x over previous
import functools

import jax
import jax.numpy as jnp
from jax import lax
from jax.experimental import pallas as pl
from jax.experimental.pallas import tpu as pltpu
from jax.experimental.pallas import tpu_sc as plsc

N_LAYERS = 4
N_P = 2048
N_K = 512
E_K = 16384
E_P = 131072

HALF_P = N_P // 2
QTR_P = N_P // 4
HALF_K = N_K // 2
EP_T = E_P // 16
EK_T = E_K // 16

TRASH_AP = QTR_P * N_P
TRASH_AK = HALF_K * N_K
TRASH_CP = HALF_P
TRASH_CK = HALF_K


def _sc_body(knn_src, knn_dst, ppi_src, ppi_dst, zf_h, of_h,
             akt_out, ck_out, ap_out, cp_out,
             aP, aK, cP, cK,
             ed, es, kd, ks,
             pidx, pcidx, kidx, kcidx,
             onesf, zf, bb, sem):
    c = lax.axis_index("c")
    s = lax.axis_index("s")

    cp_ed = pltpu.async_copy(ppi_dst.at[pl.ds(s * EP_T, EP_T)], ed, sem)
    cp_es = pltpu.async_copy(ppi_src.at[pl.ds(s * EP_T, EP_T)], es, sem)
    cp_kd = pltpu.async_copy(knn_dst.at[pl.ds(s * EK_T, EK_T)], kd, sem)
    cp_ks = pltpu.async_copy(knn_src.at[pl.ds(s * EK_T, EK_T)], ks, sem)
    pltpu.sync_copy(zf_h, zf)
    pltpu.sync_copy(of_h, onesf)

    pltpu.sync_copy(zf, aK.at[pl.ds(s * 8192, 8192)])
    pltpu.sync_copy(zf.at[pl.ds(0, 64)], cP.at[pl.ds(s * 64, 64)])
    pltpu.sync_copy(zf.at[pl.ds(0, 16)], cK.at[pl.ds(s * 16, 16)])

    @pl.when(s == 0)
    def _zero_trash():
        pltpu.sync_copy(zf.at[pl.ds(0, 128)], aK.at[pl.ds(TRASH_AK, 128)])
        pltpu.sync_copy(zf.at[pl.ds(0, 128)], cP.at[pl.ds(TRASH_CP, 128)])
        pltpu.sync_copy(zf.at[pl.ds(0, 128)], cK.at[pl.ds(TRASH_CK, 128)])

    cp_ed.wait()
    cp_es.wait()
    cp_kd.wait()
    cp_ks.wait()

    lo_cp = c * HALF_P
    lo_q0 = c * QTR_P
    lo_q1 = (2 + c) * QTR_P
    lo_k = c * HALF_K

    def pcbody(r, _):
        for j in range(8):
            off = r * 128 + j * 16
            d = ed[pl.ds(off, 16)]
            mc = (d >= lo_cp) & (d < lo_cp + HALF_P)
            pcidx[r, pl.ds(j * 16, 16)] = jnp.where(mc, d - lo_cp, TRASH_CP)
        return _
    lax.fori_loop(0, EP_T // 128, pcbody, None)

    def kbody(r, _):
        for j in range(8):
            off = r * 128 + j * 16
            d = kd[pl.ds(off, 16)]
            sv = ks[pl.ds(off, 16)]
            m1 = (sv >= lo_k) & (sv < lo_k + HALF_K)
            kidx[r, pl.ds(j * 16, 16)] = jnp.where(
                m1, (sv - lo_k) * N_K + d, TRASH_AK)
            m2 = (d >= lo_k) & (d < lo_k + HALF_K)
            kcidx[r, pl.ds(j * 16, 16)] = jnp.where(m2, d - lo_k, TRASH_CK)
        return _
    lax.fori_loop(0, EK_T // 128, kbody, None)

    plsc.subcore_barrier()

    def scatter_group(idx2d, target, nrows, chunk):
        def body(ci, _):
            base = ci * chunk
            for j in range(chunk):
                pltpu.async_copy(onesf, target.at[idx2d.at[base + j]],
                                 sem, add=True)
            for j in range(chunk):
                pltpu.make_async_copy(onesf, target.at[idx2d.at[0]],
                                      sem).wait()
            return _
        lax.fori_loop(0, nrows // chunk, body, None)

    scatter_group(pcidx, cP, EP_T // 128, 8)
    scatter_group(kidx, aK, EK_T // 128, 8)
    scatter_group(kcidx, cK, EK_T // 128, 8)

    for rnd, lo_q in enumerate((lo_q0, lo_q1)):
        def pbody(r, _, lo_q=lo_q):
            for j in range(8):
                off = r * 128 + j * 16
                d = ed[pl.ds(off, 16)]
                sv = es[pl.ds(off, 16)]
                m = (d >= lo_q) & (d < lo_q + QTR_P)
                pidx[r, pl.ds(j * 16, 16)] = jnp.where(
                    m, (d - lo_q) * N_P + sv, TRASH_AP)
            return _
        lax.fori_loop(0, EP_T // 128, pbody, None)
        for j in range(8):
            pltpu.sync_copy(zf, aP.at[pl.ds(s * 65536 + j * 8192, 8192)])
        @pl.when(s == 0)
        def _zt():
            pltpu.sync_copy(zf.at[pl.ds(0, 128)], aP.at[pl.ds(TRASH_AP, 128)])
        plsc.subcore_barrier()
        scatter_group(pidx, aP, EP_T // 128, 8)
        plsc.subcore_barrier()
        for j in range(8):
            pltpu.sync_copy(aP.at[pl.ds(s * 65536 + j * 8192, 8192)], bb)
            pltpu.sync_copy(bb, ap_out.at[pl.ds(
                (2 * rnd + c) * (QTR_P * N_P) + s * 65536 + j * 8192, 8192)])
        plsc.subcore_barrier()

    pltpu.sync_copy(aK.at[pl.ds(s * 8192, 8192)], bb)
    pltpu.sync_copy(bb, akt_out.at[pl.ds(c * (HALF_K * N_K) + s * 8192, 8192)])
    pltpu.sync_copy(cP.at[pl.ds(s * 64, 64)], bb.at[pl.ds(0, 64)])
    pltpu.sync_copy(bb.at[pl.ds(0, 64)],
                    cp_out.at[pl.ds(c * HALF_P + s * 64, 64)])
    pltpu.sync_copy(cK.at[pl.ds(s * 16, 16)], bb.at[pl.ds(64, 16)])
    pltpu.sync_copy(bb.at[pl.ds(64, 16)],
                    ck_out.at[pl.ds(c * HALF_K + s * 16, 16)])


def _sc_build(knn_src, knn_dst, ppi_src, ppi_dst):
    zf_h = jnp.zeros((8192,), jnp.float32)
    of_h = jnp.ones((128,), jnp.float32)
    mesh = plsc.VectorSubcoreMesh(core_axis_name="c", subcore_axis_name="s")
    f = pl.kernel(
        _sc_body,
        out_type=(
            jax.ShapeDtypeStruct((N_K * N_K,), jnp.float32),
            jax.ShapeDtypeStruct((N_K,), jnp.float32),
            jax.ShapeDtypeStruct((N_P * N_P,), jnp.float32),
            jax.ShapeDtypeStruct((N_P,), jnp.float32),
        ),
        mesh=mesh,
        scratch_types=(
            pltpu.VMEM_SHARED((QTR_P * N_P + 128,), jnp.float32),
            pltpu.VMEM_SHARED((HALF_K * N_K + 128,), jnp.float32),
            pltpu.VMEM_SHARED((HALF_P + 128,), jnp.float32),
            pltpu.VMEM_SHARED((HALF_K + 128,), jnp.float32),
            pltpu.VMEM((EP_T,), jnp.int32),
            pltpu.VMEM((EP_T,), jnp.int32),
            pltpu.VMEM((EK_T,), jnp.int32),
            pltpu.VMEM((EK_T,), jnp.int32),
            pltpu.VMEM((EP_T // 128, 128), jnp.int32),
            pltpu.VMEM((EP_T // 128, 128), jnp.int32),
            pltpu.VMEM((EK_T // 128, 128), jnp.int32),
            pltpu.VMEM((EK_T // 128, 128), jnp.int32),
            pltpu.VMEM((128,), jnp.float32),
            pltpu.VMEM((8192,), jnp.float32),
            pltpu.VMEM((8192,), jnp.float32),
            pltpu.SemaphoreType.DMA,
        ),
    )
    return f(knn_src, knn_dst, ppi_src, ppi_dst, zf_h, of_h)


_TILE = 256
_NT = N_P // _TILE


def _cols_kernel(e_ref, akt_ref, ck_ref, wl_ref, wr_ref, bl_ref, out_ref, mk_ref):
    t = pl.program_id(0)

    @pl.when(t == 0)
    def _():
        inv = 1.0 / jnp.maximum(ck_ref[...], 1.0)
        mk_ref[...] = jnp.dot(e_ref[...], akt_ref[...],
                              preferred_element_type=jnp.float32) * inv

    h = (jnp.dot(wl_ref[...], mk_ref[...], preferred_element_type=jnp.float32)
         + jnp.dot(wr_ref[...], e_ref[...], preferred_element_type=jnp.float32)
         + bl_ref[...])
    out_ref[...] = jnp.where(h >= 0.0, h, 0.01 * h)


def _cols_stage(e, akt, ck2d, wl, wr, bl2d):
    return pl.pallas_call(
        _cols_kernel,
        grid=(_NT,),
        in_specs=[
            pl.BlockSpec((N_P, N_K), lambda t: (0, 0)),
            pl.BlockSpec((N_K, N_K), lambda t: (0, 0)),
            pl.BlockSpec((1, N_K), lambda t: (0, 0)),
            pl.BlockSpec((_TILE, N_P), lambda t: (t, 0)),
            pl.BlockSpec((_TILE, N_P), lambda t: (t, 0)),
            pl.BlockSpec((_TILE, 1), lambda t: (t, 0)),
        ],
        out_specs=pl.BlockSpec((_TILE, N_K), lambda t: (t, 0)),
        out_shape=jax.ShapeDtypeStruct((N_P, N_K), jnp.float32),
        scratch_shapes=[pltpu.VMEM((N_P, N_K), jnp.float32)],
    )(e, akt, ck2d, wl, wr, bl2d)


def _rows_kernel(ap_ref, e1_ref, e1t_ref, cp_ref, wlt_ref, wrt_ref, rbl_ref,
                 out_ref):
    agg = jnp.dot(ap_ref[...].astype(jnp.float32), e1_ref[...],
                  preferred_element_type=jnp.float32)
    inv = 1.0 / jnp.maximum(cp_ref[...], 1.0)
    mp = agg * inv
    h = (jnp.dot(mp, wlt_ref[...], preferred_element_type=jnp.float32)
         + jnp.dot(e1t_ref[...], wrt_ref[...], preferred_element_type=jnp.float32)
         + rbl_ref[...])
    out_ref[...] = jnp.where(h >= 0.0, h, 0.01 * h)


def _rows_stage(e1, ap, cp2d, wlt, wrt, rbl2d):
    return pl.pallas_call(
        _rows_kernel,
        grid=(_NT,),
        in_specs=[
            pl.BlockSpec((_TILE, N_P), lambda t: (t, 0)),
            pl.BlockSpec((N_P, N_K), lambda t: (0, 0)),
            pl.BlockSpec((_TILE, N_K), lambda t: (t, 0)),
            pl.BlockSpec((_TILE, 1), lambda t: (t, 0)),
            pl.BlockSpec((N_K, N_K), lambda t: (0, 0)),
            pl.BlockSpec((N_K, N_K), lambda t: (0, 0)),
            pl.BlockSpec((1, N_K), lambda t: (0, 0)),
        ],
        out_specs=pl.BlockSpec((_TILE, N_K), lambda t: (t, 0)),
        out_shape=jax.ShapeDtypeStruct((N_P, N_K), jnp.float32),
    )(ap, e1, e1, cp2d, wlt, wrt, rbl2d)


def kernel(x, knn_edge_index, ppi_edge_index, cols_Wl, cols_bl, cols_Wr,
           rows_Wl, rows_bl, rows_Wr):
    akt_flat, ck, ap_flat, cp = _sc_build(
        knn_edge_index[0], knn_edge_index[1],
        ppi_edge_index[0], ppi_edge_index[1])
    akt = akt_flat.reshape(N_K, N_K)
    ap = ap_flat.reshape(N_P, N_P)
    ck2d = ck.reshape(1, N_K)
    cp2d = cp.reshape(N_P, 1)

    e = x
    for i in range(N_LAYERS):
        e = _cols_stage(e, akt, ck2d, cols_Wl[i], cols_Wr[i],
                        cols_bl[i].reshape(N_P, 1))
        e = _rows_stage(e, ap, cp2d, rows_Wl[i].T, rows_Wr[i].T,
                        rows_bl[i].reshape(1, N_K))
    return e


# speedup vs baseline: 17.6799x; 1.1900x over previous
import functools

import jax
import jax.numpy as jnp
from jax import lax
from jax.experimental import pallas as pl
from jax.experimental.pallas import tpu as pltpu
from jax.experimental.pallas import tpu_sc as plsc

N_LAYERS = 4
N_P = 2048
N_K = 512
E_K = 16384
E_P = 131072

HALF_P = N_P // 2
QTR_P = N_P // 4
HALF_K = N_K // 2
EP_T = E_P // 16
EK_T = E_K // 16

TRASH_AP = QTR_P * N_P
TRASH_AK = HALF_K * N_K
TRASH_CP = HALF_P
TRASH_CK = HALF_K


def _sc_body(knn_src, knn_dst, ppi_src, ppi_dst, zf_h, of_h,
             akt_out, ap_out,
             aP, aK,
             ed, es, kd, ks,
             pidx2d, kidx2d,
             onesf, zf, bb, sem):
    c = lax.axis_index("c")
    s = lax.axis_index("s")

    cp_ed = pltpu.async_copy(ppi_dst.at[pl.ds(s * EP_T, EP_T)], ed, sem)
    cp_es = pltpu.async_copy(ppi_src.at[pl.ds(s * EP_T, EP_T)], es, sem)
    cp_kd = pltpu.async_copy(knn_dst.at[pl.ds(s * EK_T, EK_T)], kd, sem)
    cp_ks = pltpu.async_copy(knn_src.at[pl.ds(s * EK_T, EK_T)], ks, sem)
    pltpu.sync_copy(zf_h, zf)
    pltpu.sync_copy(of_h, onesf)

    pltpu.sync_copy(zf, aK.at[pl.ds(s * 8192, 8192)])

    @pl.when(s == 0)
    def _zero_trash():
        pltpu.sync_copy(zf.at[pl.ds(0, 128)], aK.at[pl.ds(TRASH_AK, 128)])

    cp_ed.wait()
    cp_es.wait()
    cp_kd.wait()
    cp_ks.wait()

    lo_q0 = c * QTR_P
    lo_q1 = (2 + c) * QTR_P
    lo_k = c * HALF_K

    def scatter_rows(idx2d, target, nrows, csz):
        def chunk(ci, _):
            base = ci * csz
            for j in range(csz):
                pltpu.async_copy(onesf, target.at[idx2d.at[base + j]],
                                 sem, add=True)
            for j in range(csz):
                pltpu.make_async_copy(onesf, target.at[idx2d.at[0]],
                                      sem).wait()
            return _
        lax.fori_loop(0, nrows // csz, chunk, None)

    with jax.named_scope("knn_scan"):
        def kbody(r, _):
            for j in range(8):
                off = r * 128 + j * 16
                d = kd[pl.ds(off, 16)]
                sv = ks[pl.ds(off, 16)]
                m1 = (sv >= lo_k) & (sv < lo_k + HALF_K)
                kidx2d[r, pl.ds(j * 16, 16)] = jnp.where(
                    m1, (sv - lo_k) * N_K + d, TRASH_AK)
            return _
        lax.fori_loop(0, EK_T // 128, kbody, None)

    plsc.subcore_barrier()
    with jax.named_scope("knn_scatter"):
        scatter_rows(kidx2d, aK, EK_T // 128, 8)

    for rnd, lo_q in enumerate((lo_q0, lo_q1)):
        with jax.named_scope("ppi_scan"):
            def pbody(r, _, lo_q=lo_q):
                for j in range(8):
                    off = r * 128 + j * 16
                    d = ed[pl.ds(off, 16)]
                    sv = es[pl.ds(off, 16)]
                    m = (d >= lo_q) & (d < lo_q + QTR_P)
                    pidx2d[r, pl.ds(j * 16, 16)] = jnp.where(
                        m, (d - lo_q) * N_P + sv, TRASH_AP)
                return _
            lax.fori_loop(0, EP_T // 128, pbody, None)
        with jax.named_scope("ppi_zero"):
            for j in range(8):
                pltpu.sync_copy(zf, aP.at[pl.ds(s * 65536 + j * 8192, 8192)])
            @pl.when(s == 0)
            def _zt():
                pltpu.sync_copy(zf.at[pl.ds(0, 128)],
                                aP.at[pl.ds(TRASH_AP, 128)])
        plsc.subcore_barrier()
        with jax.named_scope("ppi_scatter"):
            scatter_rows(pidx2d, aP, EP_T // 128, 16)
        plsc.subcore_barrier()
        with jax.named_scope("ppi_copyout"):
            for j in range(8):
                pltpu.sync_copy(aP.at[pl.ds(s * 65536 + j * 8192, 8192)], bb)
                pltpu.sync_copy(bb, ap_out.at[pl.ds(
                    (2 * rnd + c) * (QTR_P * N_P) + s * 65536 + j * 8192,
                    8192)])
        plsc.subcore_barrier()

    with jax.named_scope("knn_copyout"):
        pltpu.sync_copy(aK.at[pl.ds(s * 8192, 8192)], bb)
        pltpu.sync_copy(bb, akt_out.at[pl.ds(c * (HALF_K * N_K) + s * 8192,
                                             8192)])


def _sc_build(knn_src, knn_dst, ppi_src, ppi_dst):
    zf_h = jnp.zeros((8192,), jnp.float32)
    of_h = jnp.ones((128,), jnp.float32)
    mesh = plsc.VectorSubcoreMesh(core_axis_name="c", subcore_axis_name="s")
    f = pl.kernel(
        _sc_body,
        out_type=(
            jax.ShapeDtypeStruct((N_K * N_K,), jnp.float32),
            jax.ShapeDtypeStruct((N_P * N_P,), jnp.float32),
        ),
        mesh=mesh,
        scratch_types=(
            pltpu.VMEM_SHARED((QTR_P * N_P + 128,), jnp.float32),
            pltpu.VMEM_SHARED((HALF_K * N_K + 128,), jnp.float32),
            pltpu.VMEM((EP_T,), jnp.int32),
            pltpu.VMEM((EP_T,), jnp.int32),
            pltpu.VMEM((EK_T,), jnp.int32),
            pltpu.VMEM((EK_T,), jnp.int32),
            pltpu.VMEM((EP_T // 128, 128), jnp.int32),
            pltpu.VMEM((EK_T // 128, 128), jnp.int32),
            pltpu.VMEM((128,), jnp.float32),
            pltpu.VMEM((8192,), jnp.float32),
            pltpu.VMEM((8192,), jnp.float32),
            pltpu.SemaphoreType.DMA,
        ),
    )
    return f(knn_src, knn_dst, ppi_src, ppi_dst, zf_h, of_h)


_TILE = 256
_NT = N_P // _TILE


def _cols_kernel(e_ref, akt_ref, wl_ref, wr_ref, bl_ref, out_ref, mk_ref):
    t = pl.program_id(0)

    @pl.when(t == 0)
    def _():
        cnt = jnp.sum(akt_ref[...], axis=0, keepdims=True)
        inv = 1.0 / jnp.maximum(cnt, 1.0)
        mk_ref[...] = jnp.dot(e_ref[...], akt_ref[...],
                              preferred_element_type=jnp.float32) * inv

    h = (jnp.dot(wl_ref[...], mk_ref[...], preferred_element_type=jnp.float32)
         + jnp.dot(wr_ref[...], e_ref[...], preferred_element_type=jnp.float32)
         + bl_ref[...])
    out_ref[...] = jnp.where(h >= 0.0, h, 0.01 * h)


def _cols_stage(e, akt, wl, wr, bl2d):
    return pl.pallas_call(
        _cols_kernel,
        grid=(_NT,),
        in_specs=[
            pl.BlockSpec((N_P, N_K), lambda t: (0, 0)),
            pl.BlockSpec((N_K, N_K), lambda t: (0, 0)),
            pl.BlockSpec((_TILE, N_P), lambda t: (t, 0)),
            pl.BlockSpec((_TILE, N_P), lambda t: (t, 0)),
            pl.BlockSpec((_TILE, 1), lambda t: (t, 0)),
        ],
        out_specs=pl.BlockSpec((_TILE, N_K), lambda t: (t, 0)),
        out_shape=jax.ShapeDtypeStruct((N_P, N_K), jnp.float32),
        scratch_shapes=[pltpu.VMEM((N_P, N_K), jnp.float32)],
    )(e, akt, wl, wr, bl2d)


def _rows_kernel(ap_ref, e1_ref, e1t_ref, wlt_ref, wrt_ref, rbl_ref,
                 out_ref):
    agg = jnp.dot(ap_ref[...], e1_ref[...],
                  preferred_element_type=jnp.float32)
    cnt = jnp.sum(ap_ref[...], axis=1, keepdims=True)
    inv = 1.0 / jnp.maximum(cnt, 1.0)
    mp = agg * inv
    h = (jnp.dot(mp, wlt_ref[...], preferred_element_type=jnp.float32)
         + jnp.dot(e1t_ref[...], wrt_ref[...], preferred_element_type=jnp.float32)
         + rbl_ref[...])
    out_ref[...] = jnp.where(h >= 0.0, h, 0.01 * h)


def _rows_stage(e1, ap, wlt, wrt, rbl2d):
    return pl.pallas_call(
        _rows_kernel,
        grid=(_NT,),
        in_specs=[
            pl.BlockSpec((_TILE, N_P), lambda t: (t, 0)),
            pl.BlockSpec((N_P, N_K), lambda t: (0, 0)),
            pl.BlockSpec((_TILE, N_K), lambda t: (t, 0)),
            pl.BlockSpec((N_K, N_K), lambda t: (0, 0)),
            pl.BlockSpec((N_K, N_K), lambda t: (0, 0)),
            pl.BlockSpec((1, N_K), lambda t: (0, 0)),
        ],
        out_specs=pl.BlockSpec((_TILE, N_K), lambda t: (t, 0)),
        out_shape=jax.ShapeDtypeStruct((N_P, N_K), jnp.float32),
    )(ap, e1, e1, wlt, wrt, rbl2d)


def kernel(x, knn_edge_index, ppi_edge_index, cols_Wl, cols_bl, cols_Wr,
           rows_Wl, rows_bl, rows_Wr):
    akt_flat, ap_flat = _sc_build(
        knn_edge_index[0], knn_edge_index[1],
        ppi_edge_index[0], ppi_edge_index[1])
    akt = akt_flat.reshape(N_K, N_K)
    ap = ap_flat.reshape(N_P, N_P)

    e = x
    for i in range(N_LAYERS):
        e = _cols_stage(e, akt, cols_Wl[i], cols_Wr[i],
                        cols_bl[i].reshape(N_P, 1))
        e = _rows_stage(e, ap, rows_Wl[i].T, rows_Wr[i].T,
                        rows_bl[i].reshape(1, N_K))
    return e


# speedup vs baseline: 17.8620x; 1.0103x over previous
import functools

import jax
import jax.numpy as jnp
from jax import lax
from jax.experimental import pallas as pl
from jax.experimental.pallas import tpu as pltpu
from jax.experimental.pallas import tpu_sc as plsc

N_LAYERS = 4
N_P = 2048
N_K = 512
E_K = 16384
E_P = 131072

HALF_P = N_P // 2
QTR_P = N_P // 4
HALF_K = N_K // 2
EP_T = E_P // 16
EK_T = E_K // 16

TRASH_AP = QTR_P * N_P
TRASH_AK = HALF_K * N_K
TRASH_CP = HALF_P
TRASH_CK = HALF_K


def _sc_body(knn_src, knn_dst, ppi_src, ppi_dst, zf_h, of_h,
             akt_out, ap_out,
             aP, aK,
             ed, es, kd, ks,
             pidx2d, kidx2d,
             onesf, zf, bb, sem):
    c = lax.axis_index("c")
    s = lax.axis_index("s")

    cp_ed = pltpu.async_copy(ppi_dst.at[pl.ds(s * EP_T, EP_T)], ed, sem)
    cp_es = pltpu.async_copy(ppi_src.at[pl.ds(s * EP_T, EP_T)], es, sem)
    cp_kd = pltpu.async_copy(knn_dst.at[pl.ds(s * EK_T, EK_T)], kd, sem)
    cp_ks = pltpu.async_copy(knn_src.at[pl.ds(s * EK_T, EK_T)], ks, sem)
    pltpu.sync_copy(zf_h, zf)
    pltpu.sync_copy(of_h, onesf)

    pltpu.sync_copy(zf, aK.at[pl.ds(s * 8192, 8192)])

    @pl.when(s == 0)
    def _zero_trash():
        pltpu.sync_copy(zf.at[pl.ds(0, 128)], aK.at[pl.ds(TRASH_AK, 128)])

    cp_ed.wait()
    cp_es.wait()
    cp_kd.wait()
    cp_ks.wait()

    lo_q0 = c * QTR_P
    lo_q1 = (2 + c) * QTR_P
    lo_k = c * HALF_K

    def scatter_rows(idx2d, target, nrows, csz):
        def chunk(ci, _):
            base = ci * csz
            for j in range(csz):
                pltpu.async_copy(onesf, target.at[idx2d.at[base + j]],
                                 sem, add=True)
            for j in range(csz):
                pltpu.make_async_copy(onesf, target.at[idx2d.at[0]],
                                      sem).wait()
            return _
        lax.fori_loop(0, nrows // csz, chunk, None)

    with jax.named_scope("knn_scan"):
        def kbody(r, _):
            for j in range(8):
                off = r * 128 + j * 16
                d = kd[pl.ds(off, 16)]
                sv = ks[pl.ds(off, 16)]
                m1 = (sv >= lo_k) & (sv < lo_k + HALF_K)
                kidx2d[r, pl.ds(j * 16, 16)] = jnp.where(
                    m1, (sv - lo_k) * N_K + d, TRASH_AK)
            return _
        lax.fori_loop(0, EK_T // 128, kbody, None)

    plsc.subcore_barrier()
    with jax.named_scope("knn_scatter"):
        scatter_rows(kidx2d, aK, EK_T // 128, 8)

    for rnd, lo_q in enumerate((lo_q0, lo_q1)):
        with jax.named_scope("ppi_scan"):
            def pbody(r, _, lo_q=lo_q):
                for j in range(8):
                    off = r * 128 + j * 16
                    d = ed[pl.ds(off, 16)]
                    sv = es[pl.ds(off, 16)]
                    m = (d >= lo_q) & (d < lo_q + QTR_P)
                    pidx2d[r, pl.ds(j * 16, 16)] = jnp.where(
                        m, (d - lo_q) * N_P + sv, TRASH_AP)
                return _
            lax.fori_loop(0, EP_T // 128, pbody, None)
        with jax.named_scope("ppi_zero"):
            for j in range(8):
                pltpu.sync_copy(zf, aP.at[pl.ds(s * 65536 + j * 8192, 8192)])
            @pl.when(s == 0)
            def _zt():
                pltpu.sync_copy(zf.at[pl.ds(0, 128)],
                                aP.at[pl.ds(TRASH_AP, 128)])
        plsc.subcore_barrier()
        with jax.named_scope("ppi_scatter"):
            scatter_rows(pidx2d, aP, EP_T // 128, 16)
        plsc.subcore_barrier()
        with jax.named_scope("ppi_copyout"):
            for j in range(8):
                pltpu.sync_copy(aP.at[pl.ds(s * 65536 + j * 8192, 8192)], bb)
                pltpu.sync_copy(bb, ap_out.at[pl.ds(
                    (2 * rnd + c) * (QTR_P * N_P) + s * 65536 + j * 8192,
                    8192)])
        plsc.subcore_barrier()

    with jax.named_scope("knn_copyout"):
        pltpu.sync_copy(aK.at[pl.ds(s * 8192, 8192)], bb)
        pltpu.sync_copy(bb, akt_out.at[pl.ds(c * (HALF_K * N_K) + s * 8192,
                                             8192)])


def _sc_build(knn_src, knn_dst, ppi_src, ppi_dst):
    zf_h = jnp.zeros((8192,), jnp.float32)
    of_h = jnp.ones((128,), jnp.float32)
    mesh = plsc.VectorSubcoreMesh(core_axis_name="c", subcore_axis_name="s")
    f = pl.kernel(
        _sc_body,
        out_type=(
            jax.ShapeDtypeStruct((N_K * N_K,), jnp.float32),
            jax.ShapeDtypeStruct((N_P * N_P,), jnp.float32),
        ),
        mesh=mesh,
        scratch_types=(
            pltpu.VMEM_SHARED((QTR_P * N_P + 128,), jnp.float32),
            pltpu.VMEM_SHARED((HALF_K * N_K + 128,), jnp.float32),
            pltpu.VMEM((EP_T,), jnp.int32),
            pltpu.VMEM((EP_T,), jnp.int32),
            pltpu.VMEM((EK_T,), jnp.int32),
            pltpu.VMEM((EK_T,), jnp.int32),
            pltpu.VMEM((EP_T // 128, 128), jnp.int32),
            pltpu.VMEM((EK_T // 128, 128), jnp.int32),
            pltpu.VMEM((128,), jnp.float32),
            pltpu.VMEM((8192,), jnp.float32),
            pltpu.VMEM((8192,), jnp.float32),
            pltpu.SemaphoreType.DMA,
        ),
    )
    return f(knn_src, knn_dst, ppi_src, ppi_dst, zf_h, of_h)


_TILE = 256
_NT = N_P // _TILE


def _cols_kernel(e_ref, akt_ref, wl_ref, wr_ref, bl_ref, out_ref,
                 mk_ref, ebf_ref):
    t = pl.program_id(0)

    @pl.when(t == 0)
    def _():
        ebf = e_ref[...].astype(jnp.bfloat16)
        ebf_ref[...] = ebf
        cnt = jnp.sum(akt_ref[...], axis=0, keepdims=True)
        inv = 1.0 / jnp.maximum(cnt, 1.0)
        akt16 = akt_ref[...].astype(jnp.bfloat16)
        mk = jnp.dot(ebf, akt16, preferred_element_type=jnp.float32) * inv
        mk_ref[...] = mk.astype(jnp.bfloat16)

    h = (jnp.dot(wl_ref[...].astype(jnp.bfloat16), mk_ref[...],
                 preferred_element_type=jnp.float32)
         + jnp.dot(wr_ref[...].astype(jnp.bfloat16), ebf_ref[...],
                   preferred_element_type=jnp.float32)
         + bl_ref[...])
    out_ref[...] = jnp.where(h >= 0.0, h, 0.01 * h)


def _cols_stage(e, akt, wl, wr, bl2d):
    return pl.pallas_call(
        _cols_kernel,
        grid=(_NT,),
        in_specs=[
            pl.BlockSpec((N_P, N_K), lambda t: (0, 0)),
            pl.BlockSpec((N_K, N_K), lambda t: (0, 0)),
            pl.BlockSpec((_TILE, N_P), lambda t: (t, 0)),
            pl.BlockSpec((_TILE, N_P), lambda t: (t, 0)),
            pl.BlockSpec((_TILE, 1), lambda t: (t, 0)),
        ],
        out_specs=pl.BlockSpec((_TILE, N_K), lambda t: (t, 0)),
        out_shape=jax.ShapeDtypeStruct((N_P, N_K), jnp.float32),
        scratch_shapes=[pltpu.VMEM((N_P, N_K), jnp.bfloat16),
                        pltpu.VMEM((N_P, N_K), jnp.bfloat16)],
    )(e, akt, wl, wr, bl2d)


def _rows_kernel(ap_ref, e1_ref, wlt_ref, wrt_ref, rbl_ref, out_ref,
                 e1bf_ref):
    t = pl.program_id(0)

    @pl.when(t == 0)
    def _():
        e1bf_ref[...] = e1_ref[...].astype(jnp.bfloat16)

    ap = ap_ref[...]
    agg = jnp.dot(ap, e1bf_ref[...], preferred_element_type=jnp.float32)
    cnt = jnp.sum(ap.astype(jnp.float32), axis=1, keepdims=True)
    inv = 1.0 / jnp.maximum(cnt, 1.0)
    mp = (agg * inv).astype(jnp.bfloat16)
    e1t = e1bf_ref[pl.ds(t * _TILE, _TILE), :]
    h = (jnp.dot(mp, wlt_ref[...].astype(jnp.bfloat16),
                 preferred_element_type=jnp.float32)
         + jnp.dot(e1t, wrt_ref[...].astype(jnp.bfloat16),
                   preferred_element_type=jnp.float32)
         + rbl_ref[...])
    out_ref[...] = jnp.where(h >= 0.0, h, 0.01 * h)


def _rows_stage(e1, ap, wlt, wrt, rbl2d):
    return pl.pallas_call(
        _rows_kernel,
        grid=(_NT,),
        in_specs=[
            pl.BlockSpec((_TILE, N_P), lambda t: (t, 0)),
            pl.BlockSpec((N_P, N_K), lambda t: (0, 0)),
            pl.BlockSpec((N_K, N_K), lambda t: (0, 0)),
            pl.BlockSpec((N_K, N_K), lambda t: (0, 0)),
            pl.BlockSpec((1, N_K), lambda t: (0, 0)),
        ],
        out_specs=pl.BlockSpec((_TILE, N_K), lambda t: (t, 0)),
        out_shape=jax.ShapeDtypeStruct((N_P, N_K), jnp.float32),
        scratch_shapes=[pltpu.VMEM((N_P, N_K), jnp.bfloat16)],
    )(ap, e1, wlt, wrt, rbl2d)


def kernel(x, knn_edge_index, ppi_edge_index, cols_Wl, cols_bl, cols_Wr,
           rows_Wl, rows_bl, rows_Wr):
    akt_flat, ap_flat = _sc_build(
        knn_edge_index[0], knn_edge_index[1],
        ppi_edge_index[0], ppi_edge_index[1])
    akt = akt_flat.reshape(N_K, N_K)
    ap = ap_flat.reshape(N_P, N_P).astype(jnp.bfloat16)

    e = x
    for i in range(N_LAYERS):
        e = _cols_stage(e, akt, cols_Wl[i], cols_Wr[i],
                        cols_bl[i].reshape(N_P, 1))
        e = _rows_stage(e, ap, rows_Wl[i].T, rows_Wr[i].T,
                        rows_bl[i].reshape(1, N_K))
    return e
